# Initial kernel scaffold; baseline (speedup 1.0000x reference)
import functools

import jax
import jax.numpy as jnp
from jax import lax
from jax.experimental import pallas as pl
from jax.experimental.pallas import tpu as pltpu
from jax.experimental.pallas import tpu_sc as plsc


def _sc_gather(item_table, user_table, hist_idx, user_idx):
    BL = hist_idx.shape[0]
    B = user_idx.shape[0]
    D = item_table.shape[1]

    info = plsc.get_sparse_core_info()
    NC, NS = info.num_cores, info.num_subcores
    NW = NC * NS

    h_per_w = BL // NW
    u_per_w = B // NW
    CH = 512
    SUB = 128
    n_chunks = h_per_w // CH

    mesh = plsc.VectorSubcoreMesh(core_axis_name="c", subcore_axis_name="s")

    @functools.partial(
        pl.kernel,
        mesh=mesh,
        out_type=[
            jax.ShapeDtypeStruct((BL, D), jnp.float32),
            jax.ShapeDtypeStruct((B, D), jnp.float32),
        ],
        scratch_types=[
            pltpu.VMEM((CH,), jnp.int32),
            pltpu.VMEM((CH, D), jnp.float32),
            pltpu.SemaphoreType.DMA,
        ],
    )
    def k(item_hbm, user_hbm, hidx_hbm, uidx_hbm, hout_hbm, uout_hbm,
          idx_v, rows_v, sem):
        wid = lax.axis_index("s") * NC + lax.axis_index("c")

        def gather_chunk(table_hbm, idx_hbm, out_hbm, base, n):
            pltpu.sync_copy(idx_hbm.at[pl.ds(base, n)], idx_v.at[pl.ds(0, n)])
            copies = []
            for j in range(n // SUB):
                copies.append(pltpu.async_copy(
                    table_hbm.at[idx_v.at[pl.ds(j * SUB, SUB)]],
                    rows_v.at[pl.ds(j * SUB, SUB)],
                    sem,
                ))
            for c in copies:
                c.wait()
            pltpu.sync_copy(rows_v.at[pl.ds(0, n)], out_hbm.at[pl.ds(base, n)])

        hbase = wid * h_per_w

        def body(i, carry):
            gather_chunk(item_hbm, hidx_hbm, hout_hbm, hbase + i * CH, CH)
            return carry

        lax.fori_loop(0, n_chunks, body, 0)

        gather_chunk(user_hbm, uidx_hbm, uout_hbm, wid * u_per_w, u_per_w)

    return k(item_table, user_table, hist_idx, user_idx)


def _dense_body(hist_ref, hid_ref, uemb_ref, uid_ref,
                wqt_ref, bq_ref, wkt_ref, bk_ref,
                w1t_ref, b1_ref, w2t_ref, b2_ref, out_ref):
    BB, L, D = hist_ref.shape
    hist = hist_ref[...]
    m = hid_ref[...] != 0
    mf = m.astype(jnp.float32)
    histm = hist * mf[..., None]

    sum_hist = jnp.sum(histm, axis=1)
    count = jnp.maximum(jnp.sum(mf, axis=1, keepdims=True), 1.0)
    query = (jnp.dot(sum_hist / count, wqt_ref[...],
                     preferred_element_type=jnp.float32) + bq_ref[...])

    keys = jnp.dot(histm.reshape(BB * L, D), wkt_ref[...],
                   preferred_element_type=jnp.float32)
    keys = keys.reshape(BB, L, D) + bk_ref[...].reshape(1, 1, D)
    scores = jnp.sum(keys * query[:, None, :], axis=2)
    scores = jnp.where(m, scores, -1e9)
    mx = jnp.max(scores, axis=1, keepdims=True)
    e = jnp.exp(scores - mx)
    w = e / jnp.sum(e, axis=1, keepdims=True)
    hist_vec = jnp.sum(histm * w[..., None], axis=1)

    uemb = uemb_ref[...] * (uid_ref[...] != 0).astype(jnp.float32)
    uv = jnp.concatenate([uemb, hist_vec], axis=1)
    h = jnp.maximum(jnp.dot(uv, w1t_ref[...],
                            preferred_element_type=jnp.float32)
                    + b1_ref[...], 0.0)
    out = (jnp.dot(h, w2t_ref[...], preferred_element_type=jnp.float32)
           + b2_ref[...])
    norm = jnp.sqrt(jnp.sum(out * out, axis=1, keepdims=True))
    out_ref[...] = out / jnp.maximum(norm, 1e-12)


def _tc_dense(hist_emb, hist_id, user_emb, user_id,
              WqT, bq, WkT, bk, W1T, b1, W2T, b2, block_b=256):
    B, L, D = hist_emb.shape
    H = W1T.shape[1]
    grid = (B // block_b,)

    full = lambda shape: pl.BlockSpec(shape, lambda i: (0,) * len(shape))
    return pl.pallas_call(
        _dense_body,
        grid=grid,
        in_specs=[
            pl.BlockSpec((block_b, L, D), lambda i: (i, 0, 0)),
            pl.BlockSpec((block_b, L), lambda i: (i, 0)),
            pl.BlockSpec((block_b, D), lambda i: (i, 0)),
            pl.BlockSpec((block_b, 1), lambda i: (i, 0)),
            full((D, D)), full((1, D)),
            full((D, D)), full((1, D)),
            full((2 * D, H)), full((1, H)),
            full((H, D)), full((1, D)),
        ],
        out_specs=pl.BlockSpec((block_b, D), lambda i: (i, 0)),
        out_shape=jax.ShapeDtypeStruct((B, D), jnp.float32),
    )(hist_emb, hist_id, user_emb, user_id,
      WqT, bq, WkT, bk, W1T, b1, W2T, b2)


def kernel(user_table, item_table, Wq, bq, Wk, bk, W1, b1, W2, b2,
           user_id, hist_article_id):
    B, L = hist_article_id.shape
    D = user_table.shape[1]

    uid = user_id.astype(jnp.int32)
    hid = hist_article_id.astype(jnp.int32)

    hist_emb, user_emb = _sc_gather(item_table, user_table,
                                    hid.reshape(B * L), uid)

    return _tc_dense(
        hist_emb.reshape(B, L, D), hid, user_emb, uid.reshape(B, 1),
        Wq.T, bq.reshape(1, D), Wk.T, bk.reshape(1, D),
        W1.T, b1.reshape(1, -1), W2.T, b2.reshape(1, D),
    )


# speedup vs baseline: 1.1122x; 1.1122x over previous
import functools

import jax
import jax.numpy as jnp
from jax import lax
from jax.experimental import pallas as pl
from jax.experimental.pallas import tpu as pltpu
from jax.experimental.pallas import tpu_sc as plsc


def _sc_gather(item_table, user_table, hist_idx, user_idx):
    BL = hist_idx.shape[0]
    B = user_idx.shape[0]
    D = item_table.shape[1]

    info = plsc.get_sparse_core_info()
    NC, NS = info.num_cores, info.num_subcores
    NW = NC * NS

    h_per_w = BL // NW
    u_per_w = B // NW
    CH = 512
    SUB = 128
    n_chunks = h_per_w // CH

    mesh = plsc.VectorSubcoreMesh(core_axis_name="c", subcore_axis_name="s")

    @functools.partial(
        pl.kernel,
        mesh=mesh,
        out_type=[
            jax.ShapeDtypeStruct((BL, D), jnp.float32),
            jax.ShapeDtypeStruct((B, D), jnp.float32),
        ],
        scratch_types=[
            pltpu.VMEM((CH,), jnp.int32),
            pltpu.VMEM((CH, D), jnp.float32),
            pltpu.SemaphoreType.DMA,
        ],
        compiler_params=pltpu.CompilerParams(use_tc_tiling_on_sc=False),
    )
    def k(item_hbm, user_hbm, hidx_hbm, uidx_hbm, hout_hbm, uout_hbm,
          idx_v, rows_v, sem):
        wid = lax.axis_index("s") * NC + lax.axis_index("c")

        def gather_chunk(table_hbm, idx_hbm, out_hbm, base, n):
            pltpu.sync_copy(idx_hbm.at[pl.ds(base, n)], idx_v.at[pl.ds(0, n)])
            copies = []
            for j in range(n // SUB):
                copies.append(pltpu.async_copy(
                    table_hbm.at[idx_v.at[pl.ds(j * SUB, SUB)]],
                    rows_v.at[pl.ds(j * SUB, SUB)],
                    sem,
                ))
            for c in copies:
                c.wait()
            pltpu.sync_copy(rows_v.at[pl.ds(0, n)], out_hbm.at[pl.ds(base, n)])

        hbase = wid * h_per_w

        def body(i, carry):
            gather_chunk(item_hbm, hidx_hbm, hout_hbm, hbase + i * CH, CH)
            return carry

        lax.fori_loop(0, n_chunks, body, 0)

        gather_chunk(user_hbm, uidx_hbm, uout_hbm, wid * u_per_w, u_per_w)

    return k(item_table, user_table, hist_idx, user_idx)


def _dense_body(hist_ref, hid_ref, uemb_ref, uid_ref,
                wqt_ref, bq_ref, wkt_ref, bk_ref,
                w1t_ref, b1_ref, w2t_ref, b2_ref, out_ref):
    BB, L, D = hist_ref.shape
    hist = hist_ref[...]
    m = hid_ref[...] != 0
    mf = m.astype(jnp.float32)
    histm = hist * mf[..., None]

    sum_hist = jnp.sum(histm, axis=1)
    count = jnp.maximum(jnp.sum(mf, axis=1, keepdims=True), 1.0)
    query = (jnp.dot(sum_hist / count, wqt_ref[...],
                     preferred_element_type=jnp.float32) + bq_ref[...])

    keys = jnp.dot(histm.reshape(BB * L, D), wkt_ref[...],
                   preferred_element_type=jnp.float32)
    keys = keys.reshape(BB, L, D) + bk_ref[...].reshape(1, 1, D)
    scores = jnp.sum(keys * query[:, None, :], axis=2)
    scores = jnp.where(m, scores, -1e9)
    mx = jnp.max(scores, axis=1, keepdims=True)
    e = jnp.exp(scores - mx)
    w = e / jnp.sum(e, axis=1, keepdims=True)
    hist_vec = jnp.sum(histm * w[..., None], axis=1)

    uemb = uemb_ref[...] * (uid_ref[...] != 0).astype(jnp.float32)
    uv = jnp.concatenate([uemb, hist_vec], axis=1)
    h = jnp.maximum(jnp.dot(uv, w1t_ref[...],
                            preferred_element_type=jnp.float32)
                    + b1_ref[...], 0.0)
    out = (jnp.dot(h, w2t_ref[...], preferred_element_type=jnp.float32)
           + b2_ref[...])
    norm = jnp.sqrt(jnp.sum(out * out, axis=1, keepdims=True))
    out_ref[...] = out / jnp.maximum(norm, 1e-12)


def _tc_dense(hist_emb, hist_id, user_emb, user_id,
              WqT, bq, WkT, bk, W1T, b1, W2T, b2, block_b=256):
    B, L, D = hist_emb.shape
    H = W1T.shape[1]
    grid = (B // block_b,)

    full = lambda shape: pl.BlockSpec(shape, lambda i: (0,) * len(shape))
    return pl.pallas_call(
        _dense_body,
        grid=grid,
        in_specs=[
            pl.BlockSpec((block_b, L, D), lambda i: (i, 0, 0)),
            pl.BlockSpec((block_b, L), lambda i: (i, 0)),
            pl.BlockSpec((block_b, D), lambda i: (i, 0)),
            pl.BlockSpec((block_b, 1), lambda i: (i, 0)),
            full((D, D)), full((1, D)),
            full((D, D)), full((1, D)),
            full((2 * D, H)), full((1, H)),
            full((H, D)), full((1, D)),
        ],
        out_specs=pl.BlockSpec((block_b, D), lambda i: (i, 0)),
        out_shape=jax.ShapeDtypeStruct((B, D), jnp.float32),
    )(hist_emb, hist_id, user_emb, user_id,
      WqT, bq, WkT, bk, W1T, b1, W2T, b2)


def kernel(user_table, item_table, Wq, bq, Wk, bk, W1, b1, W2, b2,
           user_id, hist_article_id):
    B, L = hist_article_id.shape
    D = user_table.shape[1]

    uid = user_id.astype(jnp.int32)
    hid = hist_article_id.astype(jnp.int32)

    hist_emb, user_emb = _sc_gather(item_table, user_table,
                                    hid.reshape(B * L), uid)

    return _tc_dense(
        hist_emb.reshape(B, L, D), hid, user_emb, uid.reshape(B, 1),
        Wq.T, bq.reshape(1, D), Wk.T, bk.reshape(1, D),
        W1.T, b1.reshape(1, -1), W2.T, b2.reshape(1, D),
    )
